# baseline (speedup 1.0000x reference)
import jax
import jax.numpy as jnp
from jax import lax
from jax.experimental import pallas as pl
from jax.experimental.pallas import tpu as pltpu
from jax.experimental.pallas import tpu_sc as plsc

_N = 100000
_D = 128
_S = 512
_B = 1000
_C = 400
_NCHUNK = _N // _C
_NW = 32
_ROUNDS = -(-_NCHUNK // _NW)


def _stats_body(x_ref, w1_ref, b1_ref, w2_ref, s_out, m_out, m_ref):
    i = pl.program_id(0)
    nb = pl.num_programs(0)
    neg = jnp.float32(-jnp.inf)

    @pl.when(i == 0)
    def _init():
        m_ref[...] = jnp.full(m_ref.shape, neg, jnp.float32)

    xb = x_ref[...].astype(jnp.bfloat16)
    h = jnp.maximum(
        jnp.dot(xb, w1_ref[...], preferred_element_type=jnp.float32)
        + b1_ref[...], 0.0)
    s_row = lax.dot_general(w2_ref[...], h.astype(jnp.bfloat16),
                            (((1,), (1,)), ((), ())),
                            preferred_element_type=jnp.float32)
    s_out[...] = s_row.reshape(1, 1, _B)
    m_ref[...] = jnp.maximum(m_ref[...],
                             jnp.max(s_row, axis=1, keepdims=True))

    @pl.when(i == nb - 1)
    def _fin():
        m_out[...] = jnp.broadcast_to(m_ref[...], (_S, 1))


def _stats(x, W1, b1row, w2row):
    nb = _N // _B
    return pl.pallas_call(
        _stats_body,
        grid=(nb,),
        in_specs=[
            pl.BlockSpec((_B, _D), lambda i: (i, 0)),
            pl.BlockSpec((_D, _D), lambda i: (0, 0)),
            pl.BlockSpec((1, _D), lambda i: (0, 0)),
            pl.BlockSpec((1, _D), lambda i: (0, 0)),
        ],
        out_specs=[
            pl.BlockSpec((1, 1, _B), lambda i: (i, 0, 0)),
            pl.BlockSpec((_S, 1), lambda i: (0, 0)),
        ],
        out_shape=[
            jax.ShapeDtypeStruct((nb, 1, _B), jnp.float32),
            jax.ShapeDtypeStruct((_S, 1), jnp.float32),
        ],
        scratch_shapes=[
            pltpu.VMEM((1, 1), jnp.float32),
        ],
    )(x, W1, b1row, w2row)


def _pool_body(x_hbm, s_hbm, bflat_hbm, b2d_hbm, m_hbm, out_hbm, outd_hbm,
               xva, xvb, sva, svb, wv, iva, ivb, i2va, i2vb,
               mv, dv, zv, acc, sem0, sem1):
    xbufs, sbufs, ibufs, i2bufs = (xva, xvb), (sva, svb), (iva, ivb), (i2va, i2vb)
    cid = lax.axis_index("c")
    sid = lax.axis_index("s")
    wid = sid * 2 + cid
    lane0 = lax.iota(jnp.int32, 16) == 0
    sems = (sem0, sem1)

    for r in range(32):
        for j in range(8):
            zv[r, pl.ds(j * 16, 16)] = jnp.zeros((16,), jnp.float32)
    for g in range(_S // 16):
        dv[pl.ds(g * 16, 16)] = jnp.zeros((16,), jnp.float32)
    pltpu.sync_copy(zv, acc.at[pl.ds(sid * 32, 32)])
    pltpu.sync_copy(m_hbm, mv)
    plsc.subcore_barrier()

    def start(k, slot):
        c = jnp.minimum(wid + _NW * k, _NCHUNK - 1)
        base = c * _C
        sem = sems[slot]
        return [
            pltpu.async_copy(x_hbm.at[pl.ds(base, _C)], xbufs[slot], sem),
            pltpu.async_copy(s_hbm.at[pl.ds(base, _C)], sbufs[slot], sem),
            pltpu.async_copy(bflat_hbm.at[pl.ds(base, _C)], ibufs[slot], sem),
            pltpu.async_copy(b2d_hbm.at[pl.ds(c * 4, 4)], i2bufs[slot], sem),
        ]

    descs = {0: start(0, 0)}
    for k in range(_ROUNDS):
        slot = k % 2
        for dsc in descs[k]:
            dsc.wait()
        if k + 1 < _ROUNDS:
            descs[k + 1] = start(k + 1, 1 - slot)

        xv = xbufs[slot]
        sv = sbufs[slot]
        iv = ibufs[slot]
        i2v = i2bufs[slot]
        flag = jnp.where(wid + _NW * k < _NCHUNK, 1.0, 0.0)

        def wbody(g, carry, sv=sv, iv=iv, flag=flag):
            svg = sv[pl.ds(g * 16, 16)]
            ivg = iv[pl.ds(g * 16, 16)]
            mg = plsc.load_gather(mv, [ivg])
            wv[pl.ds(g * 16, 16)] = jnp.exp(svg - mg) * flag
            return carry

        lax.fori_loop(0, _C // 16, wbody, 0)

        def rbody(i2, carry, xv=xv, iv=iv):
            for u in range(2):
                i = i2 * 2 + u
                full_i = jnp.full((16,), i, jnp.int32)
                wb = plsc.load_gather(wv, [full_i])
                sb = plsc.load_gather(iv, [full_i])
                plsc.addupdate_scatter(dv, [sb], wb, mask=lane0)
                for j in range(8):
                    xv[i, pl.ds(j * 16, 16)] = xv[i, pl.ds(j * 16, 16)] * wb
            return carry

        lax.fori_loop(0, _C // 2, rbody, 0)

        for j in range(4):
            pltpu.sync_copy(xv.at[pl.ds(j * 100, 100)],
                            acc.at[i2v.at[j]], add=True)

    pltpu.sync_copy(dv, outd_hbm.at[wid])
    plsc.subcore_barrier()
    pltpu.sync_copy(acc.at[pl.ds(sid * 32, 32)],
                    out_hbm.at[cid].at[pl.ds(sid * 32, 32)])


def _pool_sc(x, scores, bflat, b2d, m):
    mesh = plsc.VectorSubcoreMesh(
        core_axis_name="c", subcore_axis_name="s",
        num_cores=2, num_subcores=16)
    return pl.kernel(
        _pool_body,
        out_type=[
            jax.ShapeDtypeStruct((2, _S, _D), jnp.float32),
            jax.ShapeDtypeStruct((_NW, _S), jnp.float32),
        ],
        mesh=mesh,
        compiler_params=pltpu.CompilerParams(needs_layout_passes=False),
        scratch_types=[
            pltpu.VMEM((_C, _D), jnp.float32),
            pltpu.VMEM((_C, _D), jnp.float32),
            pltpu.VMEM((_C,), jnp.float32),
            pltpu.VMEM((_C,), jnp.float32),
            pltpu.VMEM((_C,), jnp.float32),
            pltpu.VMEM((_C,), jnp.int32),
            pltpu.VMEM((_C,), jnp.int32),
            pltpu.VMEM((4, _C // 4), jnp.int32),
            pltpu.VMEM((4, _C // 4), jnp.int32),
            pltpu.VMEM((_S,), jnp.float32),
            pltpu.VMEM((_S,), jnp.float32),
            pltpu.VMEM((32, _D), jnp.float32),
            pltpu.VMEM_SHARED((_S, _D), jnp.float32),
            pltpu.SemaphoreType.DMA,
            pltpu.SemaphoreType.DMA,
        ],
    )(x, scores, bflat, b2d, m)


def _finalize_body(p_ref, d_ref, out_ref):
    psum = p_ref[0] + p_ref[1]
    d = jnp.sum(d_ref[...], axis=0, keepdims=True)
    invd = jnp.where(d > 0, 1.0 / d, 0.0)
    r = lax.broadcasted_iota(jnp.int32, (_S, _S), 0)
    cc = lax.broadcasted_iota(jnp.int32, (_S, _S), 1)
    dm = jnp.where(r == cc, invd, 0.0)
    out_ref[...] = lax.dot_general(
        dm, psum, (((1,), (0,)), ((), ())),
        preferred_element_type=jnp.float32)


def _finalize(partials, d32):
    return pl.pallas_call(
        _finalize_body,
        grid=(1,),
        in_specs=[
            pl.BlockSpec((2, _S, _D), lambda i: (0, 0, 0)),
            pl.BlockSpec((_NW, _S), lambda i: (0, 0)),
        ],
        out_specs=pl.BlockSpec((_S, _D), lambda i: (0, 0)),
        out_shape=jax.ShapeDtypeStruct((_S, _D), jnp.float32),
    )(partials, d32)


def kernel(x, batch, W1, b1, W2, b2):
    bflat = batch.astype(jnp.int32)
    scores, m = _stats(x, W1.astype(jnp.bfloat16), b1.reshape(1, _D),
                       W2.astype(jnp.bfloat16).reshape(1, _D))
    partials, d32 = _pool_sc(x, scores.reshape(_N), bflat,
                             bflat.reshape(_N // 100, 100), m.reshape(_S))
    return _finalize(partials, d32)


# speedup vs baseline: 1.2582x; 1.2582x over previous
import jax
import jax.numpy as jnp
from jax import lax
from jax.experimental import pallas as pl
from jax.experimental.pallas import tpu as pltpu
from jax.experimental.pallas import tpu_sc as plsc

_N = 100000
_D = 128
_S = 512
_B = 2000
_C = 400
_NCHUNK = _N // _C
_NW = 32
_ROUNDS = -(-_NCHUNK // _NW)


def _stats_body(x_ref, w1_ref, b1_ref, w2_ref, s_out, m_out, m_ref):
    i = pl.program_id(0)
    nb = pl.num_programs(0)
    neg = jnp.float32(-jnp.inf)

    @pl.when(i == 0)
    def _init():
        m_ref[...] = jnp.full(m_ref.shape, neg, jnp.float32)

    xb = x_ref[...]
    h = jnp.maximum(
        jnp.dot(xb, w1_ref[...], preferred_element_type=jnp.float32)
        + b1_ref[...], 0.0)
    s_row = lax.dot_general(w2_ref[...], h, (((1,), (1,)), ((), ())),
                            preferred_element_type=jnp.float32)
    s_out[...] = s_row.reshape(1, 1, _B)
    m_ref[...] = jnp.maximum(m_ref[...],
                             jnp.max(s_row, axis=1, keepdims=True))

    @pl.when(i == nb - 1)
    def _fin():
        m_out[...] = jnp.broadcast_to(m_ref[...], (_S, 1))


def _stats(x, W1, b1row, w2row):
    nb = _N // _B
    return pl.pallas_call(
        _stats_body,
        grid=(nb,),
        in_specs=[
            pl.BlockSpec((_B, _D), lambda i: (i, 0)),
            pl.BlockSpec((_D, _D), lambda i: (0, 0)),
            pl.BlockSpec((1, _D), lambda i: (0, 0)),
            pl.BlockSpec((1, _D), lambda i: (0, 0)),
        ],
        out_specs=[
            pl.BlockSpec((1, 1, _B), lambda i: (i, 0, 0)),
            pl.BlockSpec((_S, 1), lambda i: (0, 0)),
        ],
        out_shape=[
            jax.ShapeDtypeStruct((nb, 1, _B), jnp.float32),
            jax.ShapeDtypeStruct((_S, 1), jnp.float32),
        ],
        scratch_shapes=[
            pltpu.VMEM((1, 1), jnp.float32),
        ],
    )(x, W1, b1row, w2row)


def _pool_body(x_hbm, s_hbm, bflat_hbm, b2d_hbm, m_hbm, out_hbm, outd_hbm,
               xva, xvb, sva, svb, wv, iva, ivb, i2va, i2vb,
               mv, dv, zv, acc, sem0, sem1):
    xbufs, sbufs, ibufs, i2bufs = (xva, xvb), (sva, svb), (iva, ivb), (i2va, i2vb)
    cid = lax.axis_index("c")
    sid = lax.axis_index("s")
    wid = sid * 2 + cid
    lane0 = lax.iota(jnp.int32, 16) == 0
    sems = (sem0, sem1)

    for r in range(32):
        for j in range(8):
            zv[r, pl.ds(j * 16, 16)] = jnp.zeros((16,), jnp.float32)
    for g in range(_S // 16):
        dv[pl.ds(g * 16, 16)] = jnp.zeros((16,), jnp.float32)
    pltpu.sync_copy(zv, acc.at[pl.ds(sid * 32, 32)])
    pltpu.sync_copy(m_hbm, mv)
    plsc.subcore_barrier()

    def start(k, slot):
        c = jnp.minimum(wid + _NW * k, _NCHUNK - 1)
        base = c * _C
        sem = sems[slot]
        return [
            pltpu.async_copy(x_hbm.at[pl.ds(base, _C)], xbufs[slot], sem),
            pltpu.async_copy(s_hbm.at[pl.ds(base, _C)], sbufs[slot], sem),
            pltpu.async_copy(bflat_hbm.at[pl.ds(base, _C)], ibufs[slot], sem),
            pltpu.async_copy(b2d_hbm.at[pl.ds(c * 4, 4)], i2bufs[slot], sem),
        ]

    descs = {0: start(0, 0)}
    for k in range(_ROUNDS):
        slot = k % 2
        for dsc in descs[k]:
            dsc.wait()
        if k + 1 < _ROUNDS:
            descs[k + 1] = start(k + 1, 1 - slot)

        xv = xbufs[slot]
        sv = sbufs[slot]
        iv = ibufs[slot]
        i2v = i2bufs[slot]
        flag = jnp.where(wid + _NW * k < _NCHUNK, 1.0, 0.0)

        def wbody(g, carry, sv=sv, iv=iv, flag=flag):
            svg = sv[pl.ds(g * 16, 16)]
            ivg = iv[pl.ds(g * 16, 16)]
            mg = plsc.load_gather(mv, [ivg])
            wv[pl.ds(g * 16, 16)] = jnp.exp(svg - mg) * flag
            return carry

        lax.fori_loop(0, _C // 16, wbody, 0)

        def rbody(i2, carry, xv=xv, iv=iv):
            for u in range(2):
                i = i2 * 2 + u
                full_i = jnp.full((16,), i, jnp.int32)
                wb = plsc.load_gather(wv, [full_i])
                sb = plsc.load_gather(iv, [full_i])
                plsc.addupdate_scatter(dv, [sb], wb, mask=lane0)
                for j in range(8):
                    xv[i, pl.ds(j * 16, 16)] = xv[i, pl.ds(j * 16, 16)] * wb
            return carry

        lax.fori_loop(0, _C // 2, rbody, 0)

        for j in range(4):
            pltpu.sync_copy(xv.at[pl.ds(j * 100, 100)],
                            acc.at[i2v.at[j]], add=True)

    pltpu.sync_copy(dv, outd_hbm.at[wid])
    plsc.subcore_barrier()
    pltpu.sync_copy(acc.at[pl.ds(sid * 32, 32)],
                    out_hbm.at[cid].at[pl.ds(sid * 32, 32)])


def _pool_sc(x, scores, bflat, b2d, m):
    mesh = plsc.VectorSubcoreMesh(
        core_axis_name="c", subcore_axis_name="s",
        num_cores=2, num_subcores=16)
    return pl.kernel(
        _pool_body,
        out_type=[
            jax.ShapeDtypeStruct((2, _S, _D), jnp.float32),
            jax.ShapeDtypeStruct((_NW, _S), jnp.float32),
        ],
        mesh=mesh,
        compiler_params=pltpu.CompilerParams(needs_layout_passes=False),
        scratch_types=[
            pltpu.VMEM((_C, _D), jnp.float32),
            pltpu.VMEM((_C, _D), jnp.float32),
            pltpu.VMEM((_C,), jnp.float32),
            pltpu.VMEM((_C,), jnp.float32),
            pltpu.VMEM((_C,), jnp.float32),
            pltpu.VMEM((_C,), jnp.int32),
            pltpu.VMEM((_C,), jnp.int32),
            pltpu.VMEM((4, _C // 4), jnp.int32),
            pltpu.VMEM((4, _C // 4), jnp.int32),
            pltpu.VMEM((_S,), jnp.float32),
            pltpu.VMEM((_S,), jnp.float32),
            pltpu.VMEM((32, _D), jnp.float32),
            pltpu.VMEM_SHARED((_S, _D), jnp.float32),
            pltpu.SemaphoreType.DMA,
            pltpu.SemaphoreType.DMA,
        ],
    )(x, scores, bflat, b2d, m)


def _finalize_body(p_ref, d_ref, out_ref):
    psum = p_ref[0] + p_ref[1]
    d = jnp.sum(d_ref[...], axis=0, keepdims=True)
    invd = jnp.where(d > 0, 1.0 / d, 0.0)
    r = lax.broadcasted_iota(jnp.int32, (_S, _S), 0)
    cc = lax.broadcasted_iota(jnp.int32, (_S, _S), 1)
    dm = jnp.where(r == cc, invd, 0.0)
    out_ref[...] = lax.dot_general(
        dm, psum, (((1,), (0,)), ((), ())),
        preferred_element_type=jnp.float32)


def _finalize(partials, d32):
    return pl.pallas_call(
        _finalize_body,
        grid=(1,),
        in_specs=[
            pl.BlockSpec((2, _S, _D), lambda i: (0, 0, 0)),
            pl.BlockSpec((_NW, _S), lambda i: (0, 0)),
        ],
        out_specs=pl.BlockSpec((_S, _D), lambda i: (0, 0)),
        out_shape=jax.ShapeDtypeStruct((_S, _D), jnp.float32),
    )(partials, d32)


def kernel(x, batch, W1, b1, W2, b2):
    bflat = batch.astype(jnp.int32)
    scores, m = _stats(x, W1, b1.reshape(1, _D), W2.reshape(1, _D))
    partials, d32 = _pool_sc(x, scores.reshape(_N), bflat,
                             bflat.reshape(_N // 100, 100), m.reshape(_S))
    return _finalize(partials, d32)


# speedup vs baseline: 1.4016x; 1.1140x over previous
import jax
import jax.numpy as jnp
from jax import lax
from jax.experimental import pallas as pl
from jax.experimental.pallas import tpu as pltpu
from jax.experimental.pallas import tpu_sc as plsc

_N = 100000
_D = 128
_S = 512
_B = 4000
_C = 400
_NCHUNK = _N // _C
_NW = 32
_ROUNDS = -(-_NCHUNK // _NW)


def _stats_body(x_ref, w1_ref, b1_ref, w2_ref, s_out, m_out, m_ref):
    i = pl.program_id(0)
    nb = pl.num_programs(0)
    neg = jnp.float32(-jnp.inf)

    @pl.when(i == 0)
    def _init():
        m_ref[...] = jnp.full(m_ref.shape, neg, jnp.float32)

    xb = x_ref[...]
    h = jnp.maximum(
        jnp.dot(xb, w1_ref[...], preferred_element_type=jnp.float32)
        + b1_ref[...], 0.0)
    s_row = lax.dot_general(w2_ref[...], h, (((1,), (1,)), ((), ())),
                            preferred_element_type=jnp.float32)
    s_out[...] = s_row.reshape(1, 1, _B)
    m_ref[...] = jnp.maximum(m_ref[...],
                             jnp.max(s_row, axis=1, keepdims=True))

    @pl.when(i == nb - 1)
    def _fin():
        m_out[...] = jnp.broadcast_to(m_ref[...], (_S, 1))


def _stats(x, W1, b1row, w2row):
    nb = _N // _B
    return pl.pallas_call(
        _stats_body,
        grid=(nb,),
        in_specs=[
            pl.BlockSpec((_B, _D), lambda i: (i, 0)),
            pl.BlockSpec((_D, _D), lambda i: (0, 0)),
            pl.BlockSpec((1, _D), lambda i: (0, 0)),
            pl.BlockSpec((1, _D), lambda i: (0, 0)),
        ],
        out_specs=[
            pl.BlockSpec((1, 1, _B), lambda i: (i, 0, 0)),
            pl.BlockSpec((_S, 1), lambda i: (0, 0)),
        ],
        out_shape=[
            jax.ShapeDtypeStruct((nb, 1, _B), jnp.float32),
            jax.ShapeDtypeStruct((_S, 1), jnp.float32),
        ],
        scratch_shapes=[
            pltpu.VMEM((1, 1), jnp.float32),
        ],
    )(x, W1, b1row, w2row)


def _pool_body(x_hbm, s_hbm, bflat_hbm, b2d_hbm, m_hbm, out_hbm, outd_hbm,
               xva, xvb, sva, svb, wv, iva, ivb, i2va, i2vb,
               mv, dv, zv, acc, sem0, sem1):
    xbufs, sbufs, ibufs, i2bufs = (xva, xvb), (sva, svb), (iva, ivb), (i2va, i2vb)
    cid = lax.axis_index("c")
    sid = lax.axis_index("s")
    wid = sid * 2 + cid
    lane0 = lax.iota(jnp.int32, 16) == 0
    sems = (sem0, sem1)

    for r in range(32):
        for j in range(8):
            zv[r, pl.ds(j * 16, 16)] = jnp.zeros((16,), jnp.float32)
    for g in range(_S // 16):
        dv[pl.ds(g * 16, 16)] = jnp.zeros((16,), jnp.float32)
    pltpu.sync_copy(zv, acc.at[pl.ds(sid * 32, 32)])
    pltpu.sync_copy(m_hbm, mv)
    plsc.subcore_barrier()

    def start(k, slot):
        c = jnp.minimum(wid + _NW * k, _NCHUNK - 1)
        base = c * _C
        sem = sems[slot]
        return [
            pltpu.async_copy(x_hbm.at[pl.ds(base, _C)], xbufs[slot], sem),
            pltpu.async_copy(s_hbm.at[pl.ds(base, _C)], sbufs[slot], sem),
            pltpu.async_copy(bflat_hbm.at[pl.ds(base, _C)], ibufs[slot], sem),
            pltpu.async_copy(b2d_hbm.at[pl.ds(c * 4, 4)], i2bufs[slot], sem),
        ]

    descs = {0: start(0, 0)}
    for k in range(_ROUNDS):
        slot = k % 2
        for dsc in descs[k]:
            dsc.wait()
        if k + 1 < _ROUNDS:
            descs[k + 1] = start(k + 1, 1 - slot)

        xv = xbufs[slot]
        sv = sbufs[slot]
        iv = ibufs[slot]
        i2v = i2bufs[slot]
        flag = jnp.where(wid + _NW * k < _NCHUNK, 1.0, 0.0)

        def wbody(g, carry, sv=sv, iv=iv, flag=flag):
            svg = sv[pl.ds(g * 16, 16)]
            ivg = iv[pl.ds(g * 16, 16)]
            mg = plsc.load_gather(mv, [ivg])
            wv[pl.ds(g * 16, 16)] = jnp.exp(svg - mg) * flag
            return carry

        lax.fori_loop(0, _C // 16, wbody, 0)

        def rbody(i2, carry, xv=xv, iv=iv):
            for u in range(2):
                i = i2 * 2 + u
                full_i = jnp.full((16,), i, jnp.int32)
                wb = plsc.load_gather(wv, [full_i])
                sb = plsc.load_gather(iv, [full_i])
                plsc.addupdate_scatter(dv, [sb], wb, mask=lane0)
                for j in range(8):
                    xv[i, pl.ds(j * 16, 16)] = xv[i, pl.ds(j * 16, 16)] * wb
            return carry

        lax.fori_loop(0, _C // 2, rbody, 0)

        for j in range(4):
            pltpu.sync_copy(xv.at[pl.ds(j * 100, 100)],
                            acc.at[i2v.at[j]], add=True)

    pltpu.sync_copy(dv, outd_hbm.at[wid])
    plsc.subcore_barrier()
    pltpu.sync_copy(acc.at[pl.ds(sid * 32, 32)],
                    out_hbm.at[cid].at[pl.ds(sid * 32, 32)])


def _pool_sc(x, scores, bflat, b2d, m):
    mesh = plsc.VectorSubcoreMesh(
        core_axis_name="c", subcore_axis_name="s",
        num_cores=2, num_subcores=16)
    return pl.kernel(
        _pool_body,
        out_type=[
            jax.ShapeDtypeStruct((2, _S, _D), jnp.float32),
            jax.ShapeDtypeStruct((_NW, _S), jnp.float32),
        ],
        mesh=mesh,
        compiler_params=pltpu.CompilerParams(needs_layout_passes=False),
        scratch_types=[
            pltpu.VMEM((_C, _D), jnp.float32),
            pltpu.VMEM((_C, _D), jnp.float32),
            pltpu.VMEM((_C,), jnp.float32),
            pltpu.VMEM((_C,), jnp.float32),
            pltpu.VMEM((_C,), jnp.float32),
            pltpu.VMEM((_C,), jnp.int32),
            pltpu.VMEM((_C,), jnp.int32),
            pltpu.VMEM((4, _C // 4), jnp.int32),
            pltpu.VMEM((4, _C // 4), jnp.int32),
            pltpu.VMEM((_S,), jnp.float32),
            pltpu.VMEM((_S,), jnp.float32),
            pltpu.VMEM((32, _D), jnp.float32),
            pltpu.VMEM_SHARED((_S, _D), jnp.float32),
            pltpu.SemaphoreType.DMA,
            pltpu.SemaphoreType.DMA,
        ],
    )(x, scores, bflat, b2d, m)


def _finalize_body(p_ref, d_ref, out_ref):
    psum = p_ref[0] + p_ref[1]
    d = jnp.sum(d_ref[...], axis=0, keepdims=True)
    invd = jnp.where(d > 0, 1.0 / d, 0.0)
    r = lax.broadcasted_iota(jnp.int32, (_S, _S), 0)
    cc = lax.broadcasted_iota(jnp.int32, (_S, _S), 1)
    dm = jnp.where(r == cc, invd, 0.0)
    out_ref[...] = lax.dot_general(
        dm, psum, (((1,), (0,)), ((), ())),
        preferred_element_type=jnp.float32)


def _finalize(partials, d32):
    return pl.pallas_call(
        _finalize_body,
        grid=(1,),
        in_specs=[
            pl.BlockSpec((2, _S, _D), lambda i: (0, 0, 0)),
            pl.BlockSpec((_NW, _S), lambda i: (0, 0)),
        ],
        out_specs=pl.BlockSpec((_S, _D), lambda i: (0, 0)),
        out_shape=jax.ShapeDtypeStruct((_S, _D), jnp.float32),
    )(partials, d32)


def kernel(x, batch, W1, b1, W2, b2):
    bflat = batch.astype(jnp.int32)
    scores, m = _stats(x, W1, b1.reshape(1, _D), W2.reshape(1, _D))
    partials, d32 = _pool_sc(x, scores.reshape(_N), bflat,
                             bflat.reshape(_N // 100, 100), m.reshape(_S))
    return _finalize(partials, d32)


# speedup vs baseline: 1.5201x; 1.0845x over previous
import jax
import jax.numpy as jnp
from jax import lax
from jax.experimental import pallas as pl
from jax.experimental.pallas import tpu as pltpu
from jax.experimental.pallas import tpu_sc as plsc

_N = 100000
_D = 128
_S = 512
_B = 10000
_C = 400
_NCHUNK = _N // _C
_NW = 32
_ROUNDS = -(-_NCHUNK // _NW)


def _stats_body(x_ref, w1_ref, b1_ref, w2_ref, s_out, m_out, m_ref):
    i = pl.program_id(0)
    nb = pl.num_programs(0)
    neg = jnp.float32(-jnp.inf)

    @pl.when(i == 0)
    def _init():
        m_ref[...] = jnp.full(m_ref.shape, neg, jnp.float32)

    xb = x_ref[...]
    h = jnp.maximum(
        jnp.dot(xb, w1_ref[...], preferred_element_type=jnp.float32)
        + b1_ref[...], 0.0)
    s_row = lax.dot_general(w2_ref[...], h, (((1,), (1,)), ((), ())),
                            preferred_element_type=jnp.float32)
    s_out[...] = s_row.reshape(1, 1, _B)
    m_ref[...] = jnp.maximum(m_ref[...],
                             jnp.max(s_row, axis=1, keepdims=True))

    @pl.when(i == nb - 1)
    def _fin():
        m_out[...] = jnp.broadcast_to(m_ref[...], (_S, 1))


def _stats(x, W1, b1row, w2row):
    nb = _N // _B
    return pl.pallas_call(
        _stats_body,
        grid=(nb,),
        in_specs=[
            pl.BlockSpec((_B, _D), lambda i: (i, 0)),
            pl.BlockSpec((_D, _D), lambda i: (0, 0)),
            pl.BlockSpec((1, _D), lambda i: (0, 0)),
            pl.BlockSpec((1, _D), lambda i: (0, 0)),
        ],
        out_specs=[
            pl.BlockSpec((1, 1, _B), lambda i: (i, 0, 0)),
            pl.BlockSpec((_S, 1), lambda i: (0, 0)),
        ],
        out_shape=[
            jax.ShapeDtypeStruct((nb, 1, _B), jnp.float32),
            jax.ShapeDtypeStruct((_S, 1), jnp.float32),
        ],
        scratch_shapes=[
            pltpu.VMEM((1, 1), jnp.float32),
        ],
    )(x, W1, b1row, w2row)


def _pool_body(x_hbm, s_hbm, bflat_hbm, b2d_hbm, m_hbm, out_hbm, outd_hbm,
               xva, xvb, sva, svb, wv, iva, ivb, i2va, i2vb,
               mv, dv, zv, acc, sem0, sem1):
    xbufs, sbufs, ibufs, i2bufs = (xva, xvb), (sva, svb), (iva, ivb), (i2va, i2vb)
    cid = lax.axis_index("c")
    sid = lax.axis_index("s")
    wid = sid * 2 + cid
    lane0 = lax.iota(jnp.int32, 16) == 0
    sems = (sem0, sem1)

    for r in range(32):
        for j in range(8):
            zv[r, pl.ds(j * 16, 16)] = jnp.zeros((16,), jnp.float32)
    for g in range(_S // 16):
        dv[pl.ds(g * 16, 16)] = jnp.zeros((16,), jnp.float32)
    pltpu.sync_copy(zv, acc.at[pl.ds(sid * 32, 32)])
    pltpu.sync_copy(m_hbm, mv)
    plsc.subcore_barrier()

    def start(k, slot):
        c = jnp.minimum(wid + _NW * k, _NCHUNK - 1)
        base = c * _C
        sem = sems[slot]
        return [
            pltpu.async_copy(x_hbm.at[pl.ds(base, _C)], xbufs[slot], sem),
            pltpu.async_copy(s_hbm.at[pl.ds(base, _C)], sbufs[slot], sem),
            pltpu.async_copy(bflat_hbm.at[pl.ds(base, _C)], ibufs[slot], sem),
            pltpu.async_copy(b2d_hbm.at[pl.ds(c * 4, 4)], i2bufs[slot], sem),
        ]

    descs = {0: start(0, 0)}
    for k in range(_ROUNDS):
        slot = k % 2
        for dsc in descs[k]:
            dsc.wait()
        if k + 1 < _ROUNDS:
            descs[k + 1] = start(k + 1, 1 - slot)

        xv = xbufs[slot]
        sv = sbufs[slot]
        iv = ibufs[slot]
        i2v = i2bufs[slot]
        flag = jnp.where(wid + _NW * k < _NCHUNK, 1.0, 0.0)

        def wbody(g, carry, sv=sv, iv=iv, flag=flag):
            svg = sv[pl.ds(g * 16, 16)]
            ivg = iv[pl.ds(g * 16, 16)]
            mg = plsc.load_gather(mv, [ivg])
            wv[pl.ds(g * 16, 16)] = jnp.exp(svg - mg) * flag
            return carry

        lax.fori_loop(0, _C // 16, wbody, 0)

        def rbody(i2, carry, xv=xv, iv=iv):
            for u in range(2):
                i = i2 * 2 + u
                full_i = jnp.full((16,), i, jnp.int32)
                wb = plsc.load_gather(wv, [full_i])
                sb = plsc.load_gather(iv, [full_i])
                plsc.addupdate_scatter(dv, [sb], wb, mask=lane0)
                for j in range(8):
                    xv[i, pl.ds(j * 16, 16)] = xv[i, pl.ds(j * 16, 16)] * wb
            return carry

        lax.fori_loop(0, _C // 2, rbody, 0)

        for j in range(4):
            pltpu.sync_copy(xv.at[pl.ds(j * 100, 100)],
                            acc.at[i2v.at[j]], add=True)

    pltpu.sync_copy(dv, outd_hbm.at[wid])
    plsc.subcore_barrier()
    pltpu.sync_copy(acc.at[pl.ds(sid * 32, 32)],
                    out_hbm.at[cid].at[pl.ds(sid * 32, 32)])


def _pool_sc(x, scores, bflat, b2d, m):
    mesh = plsc.VectorSubcoreMesh(
        core_axis_name="c", subcore_axis_name="s",
        num_cores=2, num_subcores=16)
    return pl.kernel(
        _pool_body,
        out_type=[
            jax.ShapeDtypeStruct((2, _S, _D), jnp.float32),
            jax.ShapeDtypeStruct((_NW, _S), jnp.float32),
        ],
        mesh=mesh,
        compiler_params=pltpu.CompilerParams(needs_layout_passes=False),
        scratch_types=[
            pltpu.VMEM((_C, _D), jnp.float32),
            pltpu.VMEM((_C, _D), jnp.float32),
            pltpu.VMEM((_C,), jnp.float32),
            pltpu.VMEM((_C,), jnp.float32),
            pltpu.VMEM((_C,), jnp.float32),
            pltpu.VMEM((_C,), jnp.int32),
            pltpu.VMEM((_C,), jnp.int32),
            pltpu.VMEM((4, _C // 4), jnp.int32),
            pltpu.VMEM((4, _C // 4), jnp.int32),
            pltpu.VMEM((_S,), jnp.float32),
            pltpu.VMEM((_S,), jnp.float32),
            pltpu.VMEM((32, _D), jnp.float32),
            pltpu.VMEM_SHARED((_S, _D), jnp.float32),
            pltpu.SemaphoreType.DMA,
            pltpu.SemaphoreType.DMA,
        ],
    )(x, scores, bflat, b2d, m)


def _finalize_body(p_ref, d_ref, out_ref):
    psum = p_ref[0] + p_ref[1]
    d = jnp.sum(d_ref[...], axis=0, keepdims=True)
    invd = jnp.where(d > 0, 1.0 / d, 0.0)
    r = lax.broadcasted_iota(jnp.int32, (_S, _S), 0)
    cc = lax.broadcasted_iota(jnp.int32, (_S, _S), 1)
    dm = jnp.where(r == cc, invd, 0.0)
    out_ref[...] = lax.dot_general(
        dm, psum, (((1,), (0,)), ((), ())),
        preferred_element_type=jnp.float32)


def _finalize(partials, d32):
    return pl.pallas_call(
        _finalize_body,
        grid=(1,),
        in_specs=[
            pl.BlockSpec((2, _S, _D), lambda i: (0, 0, 0)),
            pl.BlockSpec((_NW, _S), lambda i: (0, 0)),
        ],
        out_specs=pl.BlockSpec((_S, _D), lambda i: (0, 0)),
        out_shape=jax.ShapeDtypeStruct((_S, _D), jnp.float32),
    )(partials, d32)


def kernel(x, batch, W1, b1, W2, b2):
    bflat = batch.astype(jnp.int32)
    scores, m = _stats(x, W1, b1.reshape(1, _D), W2.reshape(1, _D))
    partials, d32 = _pool_sc(x, scores.reshape(_N), bflat,
                             bflat.reshape(_N // 100, 100), m.reshape(_S))
    return _finalize(partials, d32)


# speedup vs baseline: 1.5324x; 1.0081x over previous
import jax
import jax.numpy as jnp
from jax import lax
from jax.experimental import pallas as pl
from jax.experimental.pallas import tpu as pltpu
from jax.experimental.pallas import tpu_sc as plsc

_N = 100000
_D = 128
_S = 512
_B = 20000
_C = 400
_NCHUNK = _N // _C
_NW = 32
_ROUNDS = -(-_NCHUNK // _NW)


def _stats_body(x_ref, w1_ref, b1_ref, w2_ref, s_out, m_out, m_ref):
    i = pl.program_id(0)
    nb = pl.num_programs(0)
    neg = jnp.float32(-jnp.inf)

    @pl.when(i == 0)
    def _init():
        m_ref[...] = jnp.full(m_ref.shape, neg, jnp.float32)

    xb = x_ref[...]
    h = jnp.maximum(
        jnp.dot(xb, w1_ref[...], preferred_element_type=jnp.float32)
        + b1_ref[...], 0.0)
    s_row = lax.dot_general(w2_ref[...], h, (((1,), (1,)), ((), ())),
                            preferred_element_type=jnp.float32)
    s_out[...] = s_row.reshape(1, 1, _B)
    m_ref[...] = jnp.maximum(m_ref[...],
                             jnp.max(s_row, axis=1, keepdims=True))

    @pl.when(i == nb - 1)
    def _fin():
        m_out[...] = jnp.broadcast_to(m_ref[...], (_S, 1))


def _stats(x, W1, b1row, w2row):
    nb = _N // _B
    return pl.pallas_call(
        _stats_body,
        grid=(nb,),
        in_specs=[
            pl.BlockSpec((_B, _D), lambda i: (i, 0)),
            pl.BlockSpec((_D, _D), lambda i: (0, 0)),
            pl.BlockSpec((1, _D), lambda i: (0, 0)),
            pl.BlockSpec((1, _D), lambda i: (0, 0)),
        ],
        out_specs=[
            pl.BlockSpec((1, 1, _B), lambda i: (i, 0, 0)),
            pl.BlockSpec((_S, 1), lambda i: (0, 0)),
        ],
        out_shape=[
            jax.ShapeDtypeStruct((nb, 1, _B), jnp.float32),
            jax.ShapeDtypeStruct((_S, 1), jnp.float32),
        ],
        scratch_shapes=[
            pltpu.VMEM((1, 1), jnp.float32),
        ],
    )(x, W1, b1row, w2row)


def _pool_body(x_hbm, s_hbm, bflat_hbm, b2d_hbm, m_hbm, out_hbm, outd_hbm,
               xva, xvb, sva, svb, wv, iva, ivb, i2va, i2vb,
               mv, dv, zv, acc, sem0, sem1):
    xbufs, sbufs, ibufs, i2bufs = (xva, xvb), (sva, svb), (iva, ivb), (i2va, i2vb)
    cid = lax.axis_index("c")
    sid = lax.axis_index("s")
    wid = sid * 2 + cid
    lane0 = lax.iota(jnp.int32, 16) == 0
    sems = (sem0, sem1)

    for r in range(32):
        for j in range(8):
            zv[r, pl.ds(j * 16, 16)] = jnp.zeros((16,), jnp.float32)
    for g in range(_S // 16):
        dv[pl.ds(g * 16, 16)] = jnp.zeros((16,), jnp.float32)
    pltpu.sync_copy(zv, acc.at[pl.ds(sid * 32, 32)])
    pltpu.sync_copy(m_hbm, mv)
    plsc.subcore_barrier()

    def start(k, slot):
        c = jnp.minimum(wid + _NW * k, _NCHUNK - 1)
        base = c * _C
        sem = sems[slot]
        return [
            pltpu.async_copy(x_hbm.at[pl.ds(base, _C)], xbufs[slot], sem),
            pltpu.async_copy(s_hbm.at[pl.ds(base, _C)], sbufs[slot], sem),
            pltpu.async_copy(bflat_hbm.at[pl.ds(base, _C)], ibufs[slot], sem),
            pltpu.async_copy(b2d_hbm.at[pl.ds(c * 4, 4)], i2bufs[slot], sem),
        ]

    descs = {0: start(0, 0)}
    for k in range(_ROUNDS):
        slot = k % 2
        for dsc in descs[k]:
            dsc.wait()
        if k + 1 < _ROUNDS:
            descs[k + 1] = start(k + 1, 1 - slot)

        xv = xbufs[slot]
        sv = sbufs[slot]
        iv = ibufs[slot]
        i2v = i2bufs[slot]
        flag = jnp.where(wid + _NW * k < _NCHUNK, 1.0, 0.0)

        def wbody(g, carry, sv=sv, iv=iv, flag=flag):
            svg = sv[pl.ds(g * 16, 16)]
            ivg = iv[pl.ds(g * 16, 16)]
            mg = plsc.load_gather(mv, [ivg])
            wv[pl.ds(g * 16, 16)] = jnp.exp(svg - mg) * flag
            return carry

        lax.fori_loop(0, _C // 16, wbody, 0)

        def rbody(i2, carry, xv=xv, iv=iv):
            for u in range(2):
                i = i2 * 2 + u
                full_i = jnp.full((16,), i, jnp.int32)
                wb = plsc.load_gather(wv, [full_i])
                sb = plsc.load_gather(iv, [full_i])
                plsc.addupdate_scatter(dv, [sb], wb, mask=lane0)
                for j in range(8):
                    xv[i, pl.ds(j * 16, 16)] = xv[i, pl.ds(j * 16, 16)] * wb
            return carry

        lax.fori_loop(0, _C // 2, rbody, 0)

        for j in range(4):
            pltpu.sync_copy(xv.at[pl.ds(j * 100, 100)],
                            acc.at[i2v.at[j]], add=True)

    pltpu.sync_copy(dv, outd_hbm.at[wid])
    plsc.subcore_barrier()
    pltpu.sync_copy(acc.at[pl.ds(sid * 32, 32)],
                    out_hbm.at[cid].at[pl.ds(sid * 32, 32)])


def _pool_sc(x, scores, bflat, b2d, m):
    mesh = plsc.VectorSubcoreMesh(
        core_axis_name="c", subcore_axis_name="s",
        num_cores=2, num_subcores=16)
    return pl.kernel(
        _pool_body,
        out_type=[
            jax.ShapeDtypeStruct((2, _S, _D), jnp.float32),
            jax.ShapeDtypeStruct((_NW, _S), jnp.float32),
        ],
        mesh=mesh,
        compiler_params=pltpu.CompilerParams(needs_layout_passes=False),
        scratch_types=[
            pltpu.VMEM((_C, _D), jnp.float32),
            pltpu.VMEM((_C, _D), jnp.float32),
            pltpu.VMEM((_C,), jnp.float32),
            pltpu.VMEM((_C,), jnp.float32),
            pltpu.VMEM((_C,), jnp.float32),
            pltpu.VMEM((_C,), jnp.int32),
            pltpu.VMEM((_C,), jnp.int32),
            pltpu.VMEM((4, _C // 4), jnp.int32),
            pltpu.VMEM((4, _C // 4), jnp.int32),
            pltpu.VMEM((_S,), jnp.float32),
            pltpu.VMEM((_S,), jnp.float32),
            pltpu.VMEM((32, _D), jnp.float32),
            pltpu.VMEM_SHARED((_S, _D), jnp.float32),
            pltpu.SemaphoreType.DMA,
            pltpu.SemaphoreType.DMA,
        ],
    )(x, scores, bflat, b2d, m)


def _finalize_body(p_ref, d_ref, out_ref):
    psum = p_ref[0] + p_ref[1]
    d = jnp.sum(d_ref[...], axis=0, keepdims=True)
    invd = jnp.where(d > 0, 1.0 / d, 0.0)
    r = lax.broadcasted_iota(jnp.int32, (_S, _S), 0)
    cc = lax.broadcasted_iota(jnp.int32, (_S, _S), 1)
    dm = jnp.where(r == cc, invd, 0.0)
    out_ref[...] = lax.dot_general(
        dm, psum, (((1,), (0,)), ((), ())),
        preferred_element_type=jnp.float32)


def _finalize(partials, d32):
    return pl.pallas_call(
        _finalize_body,
        grid=(1,),
        in_specs=[
            pl.BlockSpec((2, _S, _D), lambda i: (0, 0, 0)),
            pl.BlockSpec((_NW, _S), lambda i: (0, 0)),
        ],
        out_specs=pl.BlockSpec((_S, _D), lambda i: (0, 0)),
        out_shape=jax.ShapeDtypeStruct((_S, _D), jnp.float32),
    )(partials, d32)


def kernel(x, batch, W1, b1, W2, b2):
    bflat = batch.astype(jnp.int32)
    scores, m = _stats(x, W1, b1.reshape(1, _D), W2.reshape(1, _D))
    partials, d32 = _pool_sc(x, scores.reshape(_N), bflat,
                             bflat.reshape(_N // 100, 100), m.reshape(_S))
    return _finalize(partials, d32)
